# Initial kernel scaffold; baseline (speedup 1.0000x reference)
import jax
import jax.numpy as jnp
from jax.experimental import pallas as pl


def kernel(x, edge_indices, W_src, W_dst, att_src, att_dst, bias_gat, W_lin, b_lin):
    raise NotImplementedError("write your pallas kernel here")


# speedup vs baseline: 20.9127x; 20.9127x over previous
import functools

import jax
import jax.numpy as jnp
from jax import lax
from jax.experimental import pallas as pl
from jax.experimental.pallas import tpu as pltpu
from jax.experimental.pallas import tpu_sc as plsc

N = 10000
D = 128
H = 128
O = 128
E = 320000
W = 144
NC = 2
NS = 16
NW = NC * NS
EPT_REAL = E // NW
K = 128
EPT = ((EPT_REAL + K - 1) // K) * K
CH = EPT // K
BN = 1000
NP = 10240
ROWS_PT = NP // NS


def _proj_body(x_ref, ws_ref, wd_ref, ats_ref, atd_ref,
               xsp_ref, as_ref, ad_ref):
    xb = x_ref[...]
    xs = jnp.dot(xb, ws_ref[...], preferred_element_type=jnp.float32,
                 precision=lax.Precision.HIGHEST)
    xd = jnp.dot(xb, wd_ref[...], preferred_element_type=jnp.float32,
                 precision=lax.Precision.HIGHEST)
    as_ref[...] = jnp.sum(xs * ats_ref[...], axis=1, keepdims=True)
    ad_ref[...] = jnp.sum(xd * atd_ref[...], axis=1, keepdims=True)
    ones = jnp.ones((BN, 1), jnp.float32)
    zeros = jnp.zeros((BN, W - H - 1), jnp.float32)
    xsp_ref[...] = jnp.concatenate([xs, ones, zeros], axis=1)


def _project(x, W_src, W_dst, att_src, att_dst):
    return pl.pallas_call(
        _proj_body,
        grid=(N // BN,),
        in_specs=[
            pl.BlockSpec((BN, D), lambda i: (i, 0)),
            pl.BlockSpec((D, H), lambda i: (0, 0)),
            pl.BlockSpec((D, H), lambda i: (0, 0)),
            pl.BlockSpec((1, H), lambda i: (0, 0)),
            pl.BlockSpec((1, H), lambda i: (0, 0)),
        ],
        out_specs=[
            pl.BlockSpec((BN, W), lambda i: (i, 0)),
            pl.BlockSpec((BN, 1), lambda i: (i, 0)),
            pl.BlockSpec((BN, 1), lambda i: (i, 0)),
        ],
        out_shape=[
            jax.ShapeDtypeStruct((N, W), jnp.float32),
            jax.ShapeDtypeStruct((N, 1), jnp.float32),
            jax.ShapeDtypeStruct((N, 1), jnp.float32),
        ],
    )(x, W_src, W_dst, att_src.reshape(1, H), att_dst.reshape(1, H))


def _edge_body(xsp_hbm, src_hbm, dst_hbm, as_hbm, ad_hbm, out_hbm,
               as_v, ad_v, src_v, dst_v, e_v, rows_v, h_sh, sem):
    c = lax.axis_index("c")
    s = lax.axis_index("s")
    wid = s * NC + c

    pltpu.sync_copy(as_hbm, as_v)
    pltpu.sync_copy(ad_hbm, ad_v)

    def _zero_row(k, carry):
        for m in range(W // 16):
            rows_v[k, pl.ds(m * 16, 16)] = jnp.zeros((16,), jnp.float32)
        return carry
    lax.fori_loop(0, K, _zero_row, 0)
    for i in range(ROWS_PT // K):
        pltpu.sync_copy(rows_v, h_sh.at[pl.ds(s * ROWS_PT + i * K, K)])
    plsc.subcore_barrier()

    base = wid * EPT

    def _chunk(ci, carry):
        off = base + ci * K
        pltpu.sync_copy(src_hbm.at[pl.ds(off, K)], src_v)
        pltpu.sync_copy(dst_hbm.at[pl.ds(off, K)], dst_v)
        gather = pltpu.async_copy(xsp_hbm.at[src_v], rows_v, sem)
        lid0 = ci * K
        for j in range(K // 16):
            s16 = src_v[pl.ds(j * 16, 16)]
            d16 = dst_v[pl.ds(j * 16, 16)]
            asg = plsc.load_gather(as_v, [s16])
            adg = plsc.load_gather(ad_v, [d16])
            al = asg + adg
            al = jnp.where(al >= 0.0, al, al * jnp.float32(0.2))
            ex = jnp.exp(al)
            lid = lid0 + j * 16 + lax.iota(jnp.int32, 16)
            ex = jnp.where(lid < EPT_REAL, ex, jnp.float32(0.0))
            e_v[pl.ds(j * 16, 16)] = ex
        gather.wait()
        for j in range(K // 16):
            e16 = e_v[pl.ds(j * 16, 16)]
            for t in range(16):
                k = j * 16 + t
                ek = e16[t]
                for m in range(W // 16):
                    rows_v[k, pl.ds(m * 16, 16)] = (
                        rows_v[k, pl.ds(m * 16, 16)] * ek)
        pltpu.sync_copy(rows_v, h_sh.at[dst_v], add=True)
        return carry

    lax.fori_loop(0, CH, _chunk, 0)

    plsc.subcore_barrier()
    for i in range(ROWS_PT // K):
        pltpu.sync_copy(h_sh.at[pl.ds(s * ROWS_PT + i * K, K)],
                        out_hbm.at[c, pl.ds(s * ROWS_PT + i * K, K)])


def _edge_pass(xsp, src_p, dst_p, a_s, a_d):
    mesh = plsc.VectorSubcoreMesh(core_axis_name="c", subcore_axis_name="s")
    f = pl.kernel(
        _edge_body,
        mesh=mesh,
        compiler_params=pltpu.CompilerParams(
            needs_layout_passes=False, use_tc_tiling_on_sc=False),
        out_type=jax.ShapeDtypeStruct((NC, NP, W), jnp.float32),
        scratch_types=[
            pltpu.VMEM((N,), jnp.float32),
            pltpu.VMEM((N,), jnp.float32),
            pltpu.VMEM((K,), jnp.int32),
            pltpu.VMEM((K,), jnp.int32),
            pltpu.VMEM((K,), jnp.float32),
            pltpu.VMEM((K, W), jnp.float32),
            pltpu.VMEM_SHARED((NP, W), jnp.float32),
            pltpu.SemaphoreType.DMA,
        ],
    )
    return f(xsp, src_p, dst_p, a_s, a_d)


def _out_body(hp_ref, bias_ref, wl_ref, bl_ref, o_ref):
    num = hp_ref[0, :, 0:H] + hp_ref[1, :, 0:H]
    den = hp_ref[0, :, H:H + 1] + hp_ref[1, :, H:H + 1]
    h = num / (den + jnp.float32(1e-16)) + bias_ref[...]
    h = jnp.maximum(h, 0.0)
    o_ref[...] = jnp.dot(h, wl_ref[...], preferred_element_type=jnp.float32,
                         precision=lax.Precision.HIGHEST) + bl_ref[...]


def _finish(hpart, bias_gat, W_lin, b_lin):
    return pl.pallas_call(
        _out_body,
        grid=(N // BN,),
        in_specs=[
            pl.BlockSpec((NC, BN, W), lambda i: (0, i, 0)),
            pl.BlockSpec((1, H), lambda i: (0, 0)),
            pl.BlockSpec((H, O), lambda i: (0, 0)),
            pl.BlockSpec((1, O), lambda i: (0, 0)),
        ],
        out_specs=pl.BlockSpec((BN, O), lambda i: (i, 0)),
        out_shape=jax.ShapeDtypeStruct((N, O), jnp.float32),
    )(hpart, bias_gat.reshape(1, H), W_lin, b_lin.reshape(1, O))


def kernel(x, edge_indices, W_src, W_dst, att_src, att_dst, bias_gat,
           W_lin, b_lin):
    src = edge_indices[0]
    dst = edge_indices[1]
    pad = jnp.zeros((NW, EPT - EPT_REAL), jnp.int32)
    src_p = jnp.concatenate([src.reshape(NW, EPT_REAL), pad], axis=1).reshape(-1)
    dst_p = jnp.concatenate([dst.reshape(NW, EPT_REAL), pad], axis=1).reshape(-1)

    xsp, a_s2, a_d2 = _project(x, W_src, W_dst, att_src, att_dst)
    hpart = _edge_pass(xsp, src_p, dst_p,
                       a_s2.reshape(N), a_d2.reshape(N))
    return _finish(hpart, bias_gat, W_lin, b_lin)
